# baseline (speedup 1.0000x reference)
import functools

import jax
import jax.numpy as jnp
from jax import lax
from jax.experimental import pallas as pl
from jax.experimental.pallas import tpu as pltpu
from jax.experimental.pallas import tpu_sc as plsc

NUM_BINS = 8192
MINV = -50.0
MAXV = 50.0
INV_WIDTH = NUM_BINS / (MAXV - MINV)
OFFSET1 = -MINV * INV_WIDTH + 1.0

N = 16777216
NC, NS, L = 2, 16, 16
NW = NC * NS
PER_W = N // NW
CHUNK = 16384
NCHUNK = PER_W // CHUNK
HPAD = NUM_BINS + 16


def _sc_hist_body(x_hbm, out_hbm, buf0, buf1, histp_v, histq_v, hist_v, sem0, sem1):
    wid = lax.axis_index("s") * NC + lax.axis_index("c")
    base = wid * PER_W
    bufs = (buf0, buf1)
    sems = (sem0, sem1)
    lane = lax.iota(jnp.int32, L)
    ones = jnp.ones((L,), jnp.float32)
    zeros = jnp.zeros((L,), jnp.float32)

    @plsc.parallel_loop(0, HPAD // L, unroll=8)
    def zbody(i):
        histp_v[pl.ds(i * L, L)] = zeros
        histq_v[pl.ds(i * L, L)] = zeros

    pltpu.async_copy(x_hbm.at[pl.ds(base, CHUNK)], buf0, sem0)
    pltpu.async_copy(x_hbm.at[pl.ds(base + CHUNK, CHUNK)], buf1, sem1)

    def process(buf):
        @plsc.parallel_loop(0, CHUNK // (2 * L), unroll=4)
        def vbody(i):
            for hist, off in ((histp_v, 0), (histq_v, L)):
                v = buf[pl.ds(i * (2 * L) + off, L)]
                t = v * INV_WIDTH + OFFSET1
                ti = t.astype(jnp.int32)
                addr = jnp.clip(jnp.where(v > MAXV, 9000, ti), 0, NUM_BINS + 2)
                plsc.addupdate_scatter(hist, [addr], ones)

    def pair_body(p, _):
        for b in range(2):
            k = p * 2 + b
            pltpu.make_async_copy(
                x_hbm.at[pl.ds(base + k * CHUNK, CHUNK)], bufs[b], sems[b]
            ).wait()
            process(bufs[b])
            @pl.when(k + 2 < NCHUNK)
            def _():
                pltpu.async_copy(
                    x_hbm.at[pl.ds(base + (k + 2) * CHUNK, CHUNK)],
                    bufs[b],
                    sems[b],
                )
        return 0

    lax.fori_loop(0, NCHUNK // 2, pair_body, 0)

    @plsc.parallel_loop(0, NUM_BINS // L, unroll=4)
    def sbody(j):
        hist_v[pl.ds(j * L, L)] = plsc.load_gather(
            histp_v, [j * L + 1 + lane]
        ) + plsc.load_gather(histq_v, [j * L + 1 + lane])

    last = hist_v[pl.ds(NUM_BINS - L, L)]
    hi = jnp.full((L,), NUM_BINS + 1, jnp.int32)
    extra = plsc.load_gather(histp_v, [hi]) + plsc.load_gather(histq_v, [hi])
    hist_v[pl.ds(NUM_BINS - L, L)] = last + jnp.where(lane == L - 1, extra, 0.0)

    pltpu.sync_copy(hist_v, out_hbm.at[wid])


_sc_hist = functools.partial(
    pl.kernel,
    out_type=jax.ShapeDtypeStruct((NW, NUM_BINS), jnp.float32),
    mesh=plsc.VectorSubcoreMesh(
        core_axis_name="c", subcore_axis_name="s", num_cores=NC, num_subcores=NS
    ),
    scratch_types=[
        pltpu.VMEM((CHUNK,), jnp.float32),
        pltpu.VMEM((CHUNK,), jnp.float32),
        pltpu.VMEM((HPAD,), jnp.float32),
        pltpu.VMEM((HPAD,), jnp.float32),
        pltpu.VMEM((NUM_BINS,), jnp.float32),
        pltpu.SemaphoreType.DMA,
        pltpu.SemaphoreType.DMA,
    ],
    compiler_params=pltpu.CompilerParams(needs_layout_passes=False),
)(_sc_hist_body)


def _merge_body(parts_ref, bins_ref, o_ref):
    o_ref[...] = jnp.sum(parts_ref[...], axis=0) + bins_ref[...]


def _merge(parts, hist_bins):
    out = pl.pallas_call(
        _merge_body,
        out_shape=jax.ShapeDtypeStruct((64, 128), jnp.float32),
    )(parts.reshape(NW, 64, 128), hist_bins.reshape(64, 128))
    return out.reshape(NUM_BINS)


def kernel(x, hist_bins):
    parts = _sc_hist(x)
    new_hist = _merge(parts, hist_bins)
    return (x, new_hist)


# speedup vs baseline: 1.0496x; 1.0496x over previous
import functools

import jax
import jax.numpy as jnp
from jax import lax
from jax.experimental import pallas as pl
from jax.experimental.pallas import tpu as pltpu
from jax.experimental.pallas import tpu_sc as plsc

NUM_BINS = 8192
MINV = -50.0
MAXV = 50.0
INV_WIDTH = NUM_BINS / (MAXV - MINV)
OFFSET1 = -MINV * INV_WIDTH + 1.0

N = 16777216
NC, NS, L = 2, 16, 16
NW = NC * NS
PER_W = N // NW
CHUNK = 32768
NCHUNK = PER_W // CHUNK
HPAD = NUM_BINS + 16


def _sc_hist_body(x_hbm, out_hbm, buf0, buf1, histp_v, hist_v, sem0, sem1):
    wid = lax.axis_index("s") * NC + lax.axis_index("c")
    base = wid * PER_W
    bufs = (buf0, buf1)
    sems = (sem0, sem1)
    lane = lax.iota(jnp.int32, L)
    ones = jnp.ones((L,), jnp.float32)
    zeros = jnp.zeros((L,), jnp.float32)

    @plsc.parallel_loop(0, HPAD // L, unroll=8)
    def zbody(i):
        histp_v[pl.ds(i * L, L)] = zeros

    pltpu.async_copy(x_hbm.at[pl.ds(base, CHUNK)], buf0, sem0)
    pltpu.async_copy(x_hbm.at[pl.ds(base + CHUNK, CHUNK)], buf1, sem1)

    def process(buf):
        @plsc.parallel_loop(0, CHUNK // L, unroll=8)
        def vbody(i):
            v = buf[pl.ds(i * L, L)]
            t = v * INV_WIDTH + OFFSET1
            ti = t.astype(jnp.int32)
            addr = jnp.clip(jnp.where(v > MAXV, 9000, ti), 0, NUM_BINS + 2)
            plsc.addupdate_scatter(histp_v, [addr], ones)

    def pair_body(p, _):
        for b in range(2):
            k = p * 2 + b
            pltpu.make_async_copy(
                x_hbm.at[pl.ds(base + k * CHUNK, CHUNK)], bufs[b], sems[b]
            ).wait()
            process(bufs[b])
            @pl.when(k + 2 < NCHUNK)
            def _():
                pltpu.async_copy(
                    x_hbm.at[pl.ds(base + (k + 2) * CHUNK, CHUNK)],
                    bufs[b],
                    sems[b],
                )
        return 0

    lax.fori_loop(0, NCHUNK // 2, pair_body, 0)

    @plsc.parallel_loop(0, NUM_BINS // L, unroll=4)
    def sbody(j):
        hist_v[pl.ds(j * L, L)] = plsc.load_gather(histp_v, [j * L + 1 + lane])

    last = hist_v[pl.ds(NUM_BINS - L, L)]
    extra = plsc.load_gather(histp_v, [jnp.full((L,), NUM_BINS + 1, jnp.int32)])
    hist_v[pl.ds(NUM_BINS - L, L)] = last + jnp.where(lane == L - 1, extra, 0.0)

    pltpu.sync_copy(hist_v, out_hbm.at[wid])


_sc_hist = functools.partial(
    pl.kernel,
    out_type=jax.ShapeDtypeStruct((NW, NUM_BINS), jnp.float32),
    mesh=plsc.VectorSubcoreMesh(
        core_axis_name="c", subcore_axis_name="s", num_cores=NC, num_subcores=NS
    ),
    scratch_types=[
        pltpu.VMEM((CHUNK,), jnp.float32),
        pltpu.VMEM((CHUNK,), jnp.float32),
        pltpu.VMEM((HPAD,), jnp.float32),
        pltpu.VMEM((NUM_BINS,), jnp.float32),
        pltpu.SemaphoreType.DMA,
        pltpu.SemaphoreType.DMA,
    ],
    compiler_params=pltpu.CompilerParams(needs_layout_passes=False),
)(_sc_hist_body)


def _merge_body(parts_ref, bins_ref, o_ref):
    o_ref[...] = jnp.sum(parts_ref[...], axis=0) + bins_ref[...]


def _merge(parts, hist_bins):
    out = pl.pallas_call(
        _merge_body,
        out_shape=jax.ShapeDtypeStruct((64, 128), jnp.float32),
    )(parts.reshape(NW, 64, 128), hist_bins.reshape(64, 128))
    return out.reshape(NUM_BINS)


def kernel(x, hist_bins):
    parts = _sc_hist(x)
    new_hist = _merge(parts, hist_bins)
    return (x, new_hist)


# speedup vs baseline: 1.0583x; 1.0083x over previous
import functools

import jax
import jax.numpy as jnp
from jax import lax
from jax.experimental import pallas as pl
from jax.experimental.pallas import tpu as pltpu
from jax.experimental.pallas import tpu_sc as plsc

NUM_BINS = 8192
MINV = -50.0
MAXV = 50.0
INV_WIDTH = NUM_BINS / (MAXV - MINV)
OFFSET1 = -MINV * INV_WIDTH + 1.0

N = 16777216
NC, NS, L = 2, 16, 16
NW = NC * NS
PER_W = N // NW
CHUNK = 16384
NCHUNK = PER_W // CHUNK
HPAD = NUM_BINS + 16


def _sc_hist_body(x_hbm, out_hbm, buf0, buf1, histp_v, hist_v, sem0, sem1):
    wid = lax.axis_index("s") * NC + lax.axis_index("c")
    base = wid * PER_W
    bufs = (buf0, buf1)
    sems = (sem0, sem1)
    lane = lax.iota(jnp.int32, L)
    ones = jnp.ones((L,), jnp.float32)
    zeros = jnp.zeros((L,), jnp.float32)

    @plsc.parallel_loop(0, HPAD // L, unroll=8)
    def zbody(i):
        histp_v[pl.ds(i * L, L)] = zeros

    pltpu.async_copy(x_hbm.at[pl.ds(base, CHUNK)], buf0, sem0)
    pltpu.async_copy(x_hbm.at[pl.ds(base + CHUNK, CHUNK)], buf1, sem1)

    def process(buf):
        @plsc.parallel_loop(0, CHUNK // L, unroll=8)
        def vbody(i):
            v = buf[pl.ds(i * L, L)]
            t = v * INV_WIDTH + OFFSET1
            ti = t.astype(jnp.int32)
            addr = jnp.clip(jnp.where(v > MAXV, 9000, ti), 0, NUM_BINS + 2)
            plsc.addupdate_scatter(histp_v, [addr], ones)

    def pair_body(p, _):
        for b in range(2):
            k = p * 2 + b
            pltpu.make_async_copy(
                x_hbm.at[pl.ds(base + k * CHUNK, CHUNK)], bufs[b], sems[b]
            ).wait()
            process(bufs[b])
            @pl.when(k + 2 < NCHUNK)
            def _():
                pltpu.async_copy(
                    x_hbm.at[pl.ds(base + (k + 2) * CHUNK, CHUNK)],
                    bufs[b],
                    sems[b],
                )
        return 0

    lax.fori_loop(0, NCHUNK // 2, pair_body, 0)

    @plsc.parallel_loop(0, NUM_BINS // L, unroll=4)
    def sbody(j):
        hist_v[pl.ds(j * L, L)] = plsc.load_gather(histp_v, [j * L + 1 + lane])

    last = hist_v[pl.ds(NUM_BINS - L, L)]
    extra = plsc.load_gather(histp_v, [jnp.full((L,), NUM_BINS + 1, jnp.int32)])
    hist_v[pl.ds(NUM_BINS - L, L)] = last + jnp.where(lane == L - 1, extra, 0.0)

    pltpu.sync_copy(hist_v, out_hbm.at[wid])


_sc_hist = functools.partial(
    pl.kernel,
    out_type=jax.ShapeDtypeStruct((NW, NUM_BINS), jnp.float32),
    mesh=plsc.VectorSubcoreMesh(
        core_axis_name="c", subcore_axis_name="s", num_cores=NC, num_subcores=NS
    ),
    scratch_types=[
        pltpu.VMEM((CHUNK,), jnp.float32),
        pltpu.VMEM((CHUNK,), jnp.float32),
        pltpu.VMEM((HPAD,), jnp.float32),
        pltpu.VMEM((NUM_BINS,), jnp.float32),
        pltpu.SemaphoreType.DMA,
        pltpu.SemaphoreType.DMA,
    ],
    compiler_params=pltpu.CompilerParams(needs_layout_passes=False),
)(_sc_hist_body)


def _merge_body(parts_ref, bins_ref, o_ref):
    o_ref[...] = jnp.sum(parts_ref[...], axis=0) + bins_ref[...]


def _merge(parts, hist_bins):
    out = pl.pallas_call(
        _merge_body,
        out_shape=jax.ShapeDtypeStruct((64, 128), jnp.float32),
    )(parts.reshape(NW, 64, 128), hist_bins.reshape(64, 128))
    return out.reshape(NUM_BINS)


def kernel(x, hist_bins):
    parts = _sc_hist(x)
    new_hist = _merge(parts, hist_bins)
    return (x, new_hist)
